# baseline (speedup 1.0000x reference)
import functools

import jax
import jax.numpy as jnp
from jax import lax
from jax.experimental import pallas as pl
from jax.experimental.pallas import tpu as pltpu
from jax.experimental.pallas import tpu_sc as plsc

N = 100000
E = 1600000
NPAD = 100352
ROWS_PER_TILE = NPAD // 16
NE_PAD = 1638400
NB = NE_PAD // 128
SB = 5
R = 1024
GRID = NPAD // R

_mesh = plsc.VectorSubcoreMesh(core_axis_name="c", subcore_axis_name="s")


@functools.partial(
    pl.kernel,
    out_type=jax.ShapeDtypeStruct((2, NPAD), jnp.float32),
    mesh=_mesh,
    scratch_types=[
        pltpu.VMEM((2, 8, 128), jnp.int32),
        pltpu.VMEM((128,), jnp.float32),
        pltpu.SemaphoreType.DMA,
        pltpu.SemaphoreType.DMA,
        pltpu.SemaphoreType.DMA,
        pltpu.SemaphoreType.DMA,
        pltpu.VMEM_SHARED((NPAD,), jnp.float32),
    ],
)
def _deg_kernel(dst2d, zeros1d, out, idx_v, ones_v,
                ssem0, ssem1, isem0, isem1, acc):
    c = lax.axis_index("c")
    s = lax.axis_index("s")
    rs = s * ROWS_PER_TILE
    ssem = (ssem0, ssem1)
    isem = (isem0, isem1)

    @pl.loop(0, 8)
    def _fill(i):
        ones_v[pl.ds(i * 16, 16)] = jnp.ones((16,), jnp.float32)

    pltpu.sync_copy(zeros1d.at[pl.ds(rs, ROWS_PER_TILE)],
                    acc.at[pl.ds(rs, ROWS_PER_TILE)])
    plsc.subcore_barrier()

    base = c * (NB // 2) + s * (NB // 32)
    n_super = NB // 32 // 8

    def _drain_ones(sem):
        pltpu.make_async_copy(zeros1d.at[pl.ds(0, 128)],
                              ones_v, sem).wait()

    def _drain_didx(sem):
        pltpu.make_async_copy(dst2d.at[pl.ds(0, 8)], idx_v.at[0], sem).wait()

    pltpu.sync_copy(dst2d.at[pl.ds(base, 8)], idx_v.at[0])

    @pl.loop(0, n_super // 2)
    def _scatter(g):
        for b in range(2):
            o = 1 - b
            nxt = g * 2 + b + 1

            if b == 0:
                @pl.when(g > 0)
                def _():
                    for j in range(8):
                        _drain_ones(ssem[o])
            else:
                for j in range(8):
                    _drain_ones(ssem[o])

            @pl.when(nxt < n_super)
            def _():
                pltpu.async_copy(dst2d.at[pl.ds(base + nxt * 8, 8)],
                                 idx_v.at[o], isem[o])

            for j in range(8):
                pltpu.async_copy(ones_v, acc.at[idx_v.at[b, j]],
                                 ssem[b], add=True)

            @pl.when(nxt < n_super)
            def _():
                _drain_didx(isem[o])

    for j in range(8):
        _drain_ones(ssem[1])

    plsc.subcore_barrier()
    pltpu.sync_copy(acc.at[pl.ds(rs, ROWS_PER_TILE)],
                    out.at[c, pl.ds(rs, ROWS_PER_TILE)])


@functools.partial(
    pl.kernel,
    out_type=(jax.ShapeDtypeStruct((NPAD, 16), jnp.float32),
              jax.ShapeDtypeStruct((NPAD, 16), jnp.float32)),
    mesh=_mesh,
    compiler_params=pltpu.CompilerParams(use_tc_tiling_on_sc=False),
    scratch_types=[
        pltpu.VMEM((2, SB * 128), jnp.int32),
        pltpu.VMEM((2, SB, 128), jnp.int32),
        pltpu.VMEM((2, SB * 128, 16), jnp.float32),
        pltpu.SemaphoreType.DMA,
        pltpu.SemaphoreType.DMA,
        pltpu.SemaphoreType.DMA,
        pltpu.SemaphoreType.DMA,
        pltpu.SemaphoreType.DMA,
        pltpu.SemaphoreType.DMA,
        pltpu.VMEM_SHARED((NPAD, 16), jnp.float32),
    ],
)
def _scatter_kernel(src1d, dst2d, ya, yb, out_a, out_b,
                    sidx, didx, rows, gsem0, gsem1, ssem0, ssem1,
                    isem0, isem1, acc):
    c = lax.axis_index("c")
    s = lax.axis_index("s")
    rs = s * ROWS_PER_TILE
    gsem = (gsem0, gsem1)
    ssem = (ssem0, ssem1)
    isem = (isem0, isem1)

    @pl.when(c == 0)
    def _():
        pltpu.sync_copy(ya.at[pl.ds(rs, ROWS_PER_TILE)],
                        acc.at[pl.ds(rs, ROWS_PER_TILE)])

    @pl.when(c == 1)
    def _():
        pltpu.sync_copy(yb.at[pl.ds(rs, ROWS_PER_TILE)],
                        acc.at[pl.ds(rs, ROWS_PER_TILE)])

    plsc.subcore_barrier()

    base = s * (NB // 16)
    n_super = NB // 16 // SB

    def _gathers(buf, sup):
        @pl.when(c == 0)
        def _():
            pltpu.async_copy(ya.at[sidx.at[buf]], rows.at[buf], gsem[buf])

        @pl.when(c == 1)
        def _():
            pltpu.async_copy(yb.at[sidx.at[buf]], rows.at[buf], gsem[buf])

    def _drain_rows(sem):
        pltpu.make_async_copy(ya.at[sidx.at[0]], rows.at[0], sem).wait()

    def _drain_sidx(sem):
        pltpu.make_async_copy(src1d.at[pl.ds(0, SB * 128)], sidx.at[0],
                              sem).wait()

    def _drain_didx(sem):
        pltpu.make_async_copy(dst2d.at[pl.ds(0, SB)], didx.at[0], sem).wait()

    def _drain_8k(sem):
        pltpu.make_async_copy(ya.at[pl.ds(0, 128)],
                              rows.at[0, pl.ds(0, 128)], sem).wait()

    pltpu.sync_copy(src1d.at[pl.ds(base * 128, SB * 128)], sidx.at[0])
    pltpu.sync_copy(dst2d.at[pl.ds(base, SB)], didx.at[0])
    _gathers(0, 0)

    @pl.loop(0, n_super // 2)
    def _body(g):
        for b in range(2):
            o = 1 - b
            nxt = g * 2 + b + 1

            if b == 0:
                @pl.when(g > 0)
                def _():
                    for j in range(SB):
                        _drain_8k(ssem[o])
            else:
                for j in range(SB):
                    _drain_8k(ssem[o])

            @pl.when(nxt < n_super)
            def _():
                pltpu.async_copy(
                    src1d.at[pl.ds((base + nxt * SB) * 128, SB * 128)],
                    sidx.at[o], isem[o])
                pltpu.async_copy(dst2d.at[pl.ds(base + nxt * SB, SB)],
                                 didx.at[o], isem[o])

            _drain_rows(gsem[b])

            for j in range(SB):
                pltpu.async_copy(rows.at[b, pl.ds(j * 128, 128)],
                                 acc.at[didx.at[b, j]], ssem[b], add=True)

            @pl.when(nxt < n_super)
            def _():
                _drain_sidx(isem[o])
                _drain_didx(isem[o])
                _gathers(o, nxt)

    for j in range(SB):
        _drain_8k(ssem[1])

    plsc.subcore_barrier()

    @pl.when(c == 0)
    def _():
        pltpu.sync_copy(acc.at[pl.ds(rs, ROWS_PER_TILE)],
                        out_a.at[pl.ds(rs, ROWS_PER_TILE)])

    @pl.when(c == 1)
    def _():
        pltpu.sync_copy(acc.at[pl.ds(rs, ROWS_PER_TILE)],
                        out_b.at[pl.ds(rs, ROWS_PER_TILE)])


def _prep_body(deg_ref, x_ref, ya_ref, yb_ref, dinv_ref):
    deg = deg_ref[0, :] + deg_ref[1, :] + 1.0
    dinv = lax.rsqrt(deg)
    dinv = dinv * (1.5 - 0.5 * deg * dinv * dinv)
    y = x_ref[...] * dinv[:, None]
    ya_ref[...] = y[:, :16]
    yb_ref[...] = y[:, 16:]
    dinv_ref[...] = dinv[:, None]


_prep_call = pl.pallas_call(
    _prep_body,
    grid=(GRID,),
    in_specs=[
        pl.BlockSpec((2, R), lambda i: (0, i)),
        pl.BlockSpec((R, 32), lambda i: (i, 0)),
    ],
    out_specs=[
        pl.BlockSpec((R, 16), lambda i: (i, 0)),
        pl.BlockSpec((R, 16), lambda i: (i, 0)),
        pl.BlockSpec((R, 1), lambda i: (i, 0)),
    ],
    out_shape=[
        jax.ShapeDtypeStruct((NPAD, 16), jnp.float32),
        jax.ShapeDtypeStruct((NPAD, 16), jnp.float32),
        jax.ShapeDtypeStruct((NPAD, 1), jnp.float32),
    ],
)


def _l1_body(s0_ref, s1_ref, dv_ref, w1a_ref, w1b_ref, b1_ref, w2_ref,
             ya_ref, yb_ref):
    d = dv_ref[...]
    h = (jnp.dot(s0_ref[...] * d, w1a_ref[...],
                 preferred_element_type=jnp.float32, precision=lax.Precision.HIGHEST)
         + jnp.dot(s1_ref[...] * d, w1b_ref[...],
                   preferred_element_type=jnp.float32, precision=lax.Precision.HIGHEST)
         + b1_ref[...])
    h = jnp.maximum(h, 0.0)
    g = jnp.dot(h, w2_ref[...], preferred_element_type=jnp.float32, precision=lax.Precision.HIGHEST) * d
    ya_ref[...] = g[:, :16]
    yb_ref[...] = g[:, 16:]


_l1_call = pl.pallas_call(
    _l1_body,
    grid=(GRID,),
    in_specs=[
        pl.BlockSpec((R, 16), lambda i: (i, 0)),
        pl.BlockSpec((R, 16), lambda i: (i, 0)),
        pl.BlockSpec((R, 1), lambda i: (i, 0)),
        pl.BlockSpec((16, 64), lambda i: (0, 0)),
        pl.BlockSpec((16, 64), lambda i: (0, 0)),
        pl.BlockSpec((1, 64), lambda i: (0, 0)),
        pl.BlockSpec((64, 32), lambda i: (0, 0)),
    ],
    out_specs=[
        pl.BlockSpec((R, 16), lambda i: (i, 0)),
        pl.BlockSpec((R, 16), lambda i: (i, 0)),
    ],
    out_shape=[
        jax.ShapeDtypeStruct((NPAD, 16), jnp.float32),
        jax.ShapeDtypeStruct((NPAD, 16), jnp.float32),
    ],
)


def _head_body(s0_ref, s1_ref, dv_ref, b2a_ref, b2b_ref, w3a_ref, w3b_ref,
               b3_ref, out_ref):
    d = dv_ref[...]
    h0 = jnp.maximum(s0_ref[...] * d + b2a_ref[...], 0.0)
    h1 = jnp.maximum(s1_ref[...] * d + b2b_ref[...], 0.0)
    out_ref[...] = (jnp.dot(h0, w3a_ref[...],
                            preferred_element_type=jnp.float32, precision=lax.Precision.HIGHEST)
                    + jnp.dot(h1, w3b_ref[...],
                              preferred_element_type=jnp.float32, precision=lax.Precision.HIGHEST)
                    + b3_ref[...])


_head_call = pl.pallas_call(
    _head_body,
    grid=(GRID,),
    in_specs=[
        pl.BlockSpec((R, 16), lambda i: (i, 0)),
        pl.BlockSpec((R, 16), lambda i: (i, 0)),
        pl.BlockSpec((R, 1), lambda i: (i, 0)),
        pl.BlockSpec((1, 16), lambda i: (0, 0)),
        pl.BlockSpec((1, 16), lambda i: (0, 0)),
        pl.BlockSpec((16, 1), lambda i: (0, 0)),
        pl.BlockSpec((16, 1), lambda i: (0, 0)),
        pl.BlockSpec((1, 1), lambda i: (0, 0)),
    ],
    out_specs=pl.BlockSpec((R, 1), lambda i: (i, 0)),
    out_shape=jax.ShapeDtypeStruct((NPAD, 1), jnp.float32),
)


def kernel(x, edge_index, W1, b1, W2, b2, W3, b3):
    src = edge_index[0].astype(jnp.int32)
    dst = edge_index[1].astype(jnp.int32)
    pad = jnp.full((NE_PAD - E,), NPAD - 1, jnp.int32)
    src2d = jnp.concatenate([src, pad]).reshape(NB, 128)
    dst2d = jnp.concatenate([dst, pad]).reshape(NB, 128)

    xpad = jnp.pad(x, ((0, NPAD - N), (0, 32 - x.shape[1])))
    zeros1d = jnp.zeros((NPAD,), jnp.float32)

    src1d = jnp.concatenate([src, pad])
    deg2 = _deg_kernel(dst2d, zeros1d)
    ya, yb, dinv = _prep_call(deg2, xpad)

    s0, s1 = _scatter_kernel(src1d, dst2d, ya, yb)

    w1p = jnp.pad(W1, ((0, 32 - W1.shape[0]), (0, 0)))
    y2a, y2b = _l1_call(s0, s1, dinv, w1p[:16], w1p[16:], b1[None, :], W2)

    t0, t1 = _scatter_kernel(src1d, dst2d, y2a, y2b)

    out = _head_call(t0, t1, dinv, b2[None, :16], b2[None, 16:],
                     W3[:16], W3[16:], b3[None, :])
    return out[:N]


# speedup vs baseline: 1.0261x; 1.0261x over previous
import functools

import jax
import jax.numpy as jnp
from jax import lax
from jax.experimental import pallas as pl
from jax.experimental.pallas import tpu as pltpu
from jax.experimental.pallas import tpu_sc as plsc

N = 100000
E = 1600000
NPAD = 100352
ROWS_PER_TILE = NPAD // 16
NE_PAD = 1638400
NB = NE_PAD // 128
SB = 5
R = 1024
GRID = NPAD // R

_mesh = plsc.VectorSubcoreMesh(core_axis_name="c", subcore_axis_name="s")


@functools.partial(
    pl.kernel,
    out_type=jax.ShapeDtypeStruct((2, NPAD), jnp.float32),
    mesh=_mesh,
    scratch_types=[
        pltpu.VMEM((2, 8, 128), jnp.int32),
        pltpu.VMEM((128,), jnp.float32),
        pltpu.SemaphoreType.DMA,
        pltpu.SemaphoreType.DMA,
        pltpu.SemaphoreType.DMA,
        pltpu.SemaphoreType.DMA,
        pltpu.VMEM_SHARED((NPAD,), jnp.float32),
    ],
)
def _deg_kernel(dst2d, zeros1d, out, idx_v, ones_v,
                ssem0, ssem1, isem0, isem1, acc):
    c = lax.axis_index("c")
    s = lax.axis_index("s")
    rs = s * ROWS_PER_TILE
    ssem = (ssem0, ssem1)
    isem = (isem0, isem1)

    @pl.loop(0, 8)
    def _fill(i):
        ones_v[pl.ds(i * 16, 16)] = jnp.ones((16,), jnp.float32)

    pltpu.sync_copy(zeros1d.at[pl.ds(rs, ROWS_PER_TILE)],
                    acc.at[pl.ds(rs, ROWS_PER_TILE)])
    plsc.subcore_barrier()

    base = c * (NB // 2) + s * (NB // 32)
    n_super = NB // 32 // 8

    def _drain_ones(sem):
        pltpu.make_async_copy(zeros1d.at[pl.ds(0, 128)],
                              ones_v, sem).wait()

    def _drain_didx(sem):
        pltpu.make_async_copy(dst2d.at[pl.ds(0, 8)], idx_v.at[0], sem).wait()

    pltpu.sync_copy(dst2d.at[pl.ds(base, 8)], idx_v.at[0])

    @pl.loop(0, n_super // 2)
    def _scatter(g):
        for b in range(2):
            o = 1 - b
            nxt = g * 2 + b + 1

            if b == 0:
                @pl.when(g > 0)
                def _():
                    for j in range(8):
                        _drain_ones(ssem[o])
            else:
                for j in range(8):
                    _drain_ones(ssem[o])

            @pl.when(nxt < n_super)
            def _():
                pltpu.async_copy(dst2d.at[pl.ds(base + nxt * 8, 8)],
                                 idx_v.at[o], isem[o])

            for j in range(8):
                pltpu.async_copy(ones_v, acc.at[idx_v.at[b, j]],
                                 ssem[b], add=True)

            @pl.when(nxt < n_super)
            def _():
                _drain_didx(isem[o])

    for j in range(8):
        _drain_ones(ssem[1])

    plsc.subcore_barrier()
    pltpu.sync_copy(acc.at[pl.ds(rs, ROWS_PER_TILE)],
                    out.at[c, pl.ds(rs, ROWS_PER_TILE)])


@functools.partial(
    pl.kernel,
    out_type=(jax.ShapeDtypeStruct((NPAD, 16), jnp.float32),
              jax.ShapeDtypeStruct((NPAD, 16), jnp.float32)),
    mesh=_mesh,
    compiler_params=pltpu.CompilerParams(use_tc_tiling_on_sc=False),
    scratch_types=[
        pltpu.VMEM((2, SB * 128), jnp.int32),
        pltpu.VMEM((2, SB, 128), jnp.int32),
        pltpu.VMEM((2, SB * 128, 16), jnp.float32),
        pltpu.SemaphoreType.DMA,
        pltpu.SemaphoreType.DMA,
        pltpu.SemaphoreType.DMA,
        pltpu.SemaphoreType.DMA,
        pltpu.SemaphoreType.DMA,
        pltpu.SemaphoreType.DMA,
        pltpu.VMEM_SHARED((NPAD, 16), jnp.float32),
    ],
)
def _scatter_kernel(src1d, dst2d, ya, yb, out_a, out_b,
                    sidx, didx, rows, gsem0, gsem1, ssem0, ssem1,
                    isem0, isem1, acc):
    c = lax.axis_index("c")
    s = lax.axis_index("s")
    rs = s * ROWS_PER_TILE
    gsem = (gsem0, gsem1)
    ssem = (ssem0, ssem1)
    isem = (isem0, isem1)

    @pl.when(c == 0)
    def _():
        pltpu.sync_copy(ya.at[pl.ds(rs, ROWS_PER_TILE)],
                        acc.at[pl.ds(rs, ROWS_PER_TILE)])

    @pl.when(c == 1)
    def _():
        pltpu.sync_copy(yb.at[pl.ds(rs, ROWS_PER_TILE)],
                        acc.at[pl.ds(rs, ROWS_PER_TILE)])

    plsc.subcore_barrier()

    base = s * (NB // 16)
    n_super = NB // 16 // SB

    def _gathers(buf, sup):
        @pl.when(c == 0)
        def _():
            pltpu.async_copy(ya.at[sidx.at[buf]], rows.at[buf], gsem[buf])

        @pl.when(c == 1)
        def _():
            pltpu.async_copy(yb.at[sidx.at[buf]], rows.at[buf], gsem[buf])

    def _drain_rows(sem):
        pltpu.make_async_copy(ya.at[sidx.at[0]], rows.at[0], sem).wait()

    def _drain_sidx(sem):
        pltpu.make_async_copy(src1d.at[pl.ds(0, SB * 128)], sidx.at[0],
                              sem).wait()

    def _drain_didx(sem):
        pltpu.make_async_copy(dst2d.at[pl.ds(0, SB)], didx.at[0], sem).wait()

    def _drain_8k(sem):
        pltpu.make_async_copy(ya.at[pl.ds(0, 128)],
                              rows.at[0, pl.ds(0, 128)], sem).wait()

    pltpu.sync_copy(src1d.at[pl.ds(base * 128, SB * 128)], sidx.at[0])
    pltpu.sync_copy(dst2d.at[pl.ds(base, SB)], didx.at[0])
    _gathers(0, 0)

    @pl.loop(0, n_super // 2)
    def _body(g):
        for b in range(2):
            o = 1 - b
            nxt = g * 2 + b + 1

            if b == 0:
                @pl.when(g > 0)
                def _():
                    for j in range(SB):
                        _drain_8k(ssem[o])
            else:
                for j in range(SB):
                    _drain_8k(ssem[o])

            @pl.when(nxt < n_super)
            def _():
                pltpu.async_copy(
                    src1d.at[pl.ds((base + nxt * SB) * 128, SB * 128)],
                    sidx.at[o], isem[o])
                pltpu.async_copy(dst2d.at[pl.ds(base + nxt * SB, SB)],
                                 didx.at[o], isem[o])

            _drain_rows(gsem[b])

            for j in range(SB):
                pltpu.async_copy(rows.at[b, pl.ds(j * 128, 128)],
                                 acc.at[didx.at[b, j]], ssem[b], add=True)

            @pl.when(nxt < n_super)
            def _():
                _drain_sidx(isem[o])
                _drain_didx(isem[o])
                _gathers(o, nxt)

    for j in range(SB):
        _drain_8k(ssem[1])

    plsc.subcore_barrier()

    @pl.when(c == 0)
    def _():
        pltpu.sync_copy(acc.at[pl.ds(rs, ROWS_PER_TILE)],
                        out_a.at[pl.ds(rs, ROWS_PER_TILE)])

    @pl.when(c == 1)
    def _():
        pltpu.sync_copy(acc.at[pl.ds(rs, ROWS_PER_TILE)],
                        out_b.at[pl.ds(rs, ROWS_PER_TILE)])


def _dinv_of(deg_ref):
    deg = deg_ref[0, :] + deg_ref[1, :] + 1.0
    dinv = lax.rsqrt(deg)
    return dinv * (1.5 - 0.5 * deg * dinv * dinv)


def _prep_body(deg_ref, x_ref, ya_ref, yb_ref):
    dinv = _dinv_of(deg_ref)
    y = x_ref[...] * dinv[:, None]
    ya_ref[...] = y[:, :16]
    yb_ref[...] = y[:, 16:]


_prep_call = pl.pallas_call(
    _prep_body,
    grid=(GRID,),
    in_specs=[
        pl.BlockSpec((2, R), lambda i: (0, i)),
        pl.BlockSpec((R, 32), lambda i: (i, 0)),
    ],
    out_specs=[
        pl.BlockSpec((R, 16), lambda i: (i, 0)),
        pl.BlockSpec((R, 16), lambda i: (i, 0)),
    ],
    out_shape=[
        jax.ShapeDtypeStruct((NPAD, 16), jnp.float32),
        jax.ShapeDtypeStruct((NPAD, 16), jnp.float32),
    ],
)


def _l1_body(s0_ref, s1_ref, deg_ref, w1a_ref, w1b_ref, b1_ref, w2_ref,
             ya_ref, yb_ref):
    d = _dinv_of(deg_ref)[:, None]
    h = (jnp.dot(s0_ref[...] * d, w1a_ref[...],
                 preferred_element_type=jnp.float32, precision=lax.Precision.HIGHEST)
         + jnp.dot(s1_ref[...] * d, w1b_ref[...],
                   preferred_element_type=jnp.float32, precision=lax.Precision.HIGHEST)
         + b1_ref[...])
    h = jnp.maximum(h, 0.0)
    g = jnp.dot(h, w2_ref[...], preferred_element_type=jnp.float32, precision=lax.Precision.HIGHEST) * d
    ya_ref[...] = g[:, :16]
    yb_ref[...] = g[:, 16:]


_l1_call = pl.pallas_call(
    _l1_body,
    grid=(GRID,),
    in_specs=[
        pl.BlockSpec((R, 16), lambda i: (i, 0)),
        pl.BlockSpec((R, 16), lambda i: (i, 0)),
        pl.BlockSpec((2, R), lambda i: (0, i)),
        pl.BlockSpec((16, 64), lambda i: (0, 0)),
        pl.BlockSpec((16, 64), lambda i: (0, 0)),
        pl.BlockSpec((1, 64), lambda i: (0, 0)),
        pl.BlockSpec((64, 32), lambda i: (0, 0)),
    ],
    out_specs=[
        pl.BlockSpec((R, 16), lambda i: (i, 0)),
        pl.BlockSpec((R, 16), lambda i: (i, 0)),
    ],
    out_shape=[
        jax.ShapeDtypeStruct((NPAD, 16), jnp.float32),
        jax.ShapeDtypeStruct((NPAD, 16), jnp.float32),
    ],
)


def _head_body(s0_ref, s1_ref, deg_ref, b2a_ref, b2b_ref, w3a_ref, w3b_ref,
               b3_ref, out_ref):
    d = _dinv_of(deg_ref)[:, None]
    h0 = jnp.maximum(s0_ref[...] * d + b2a_ref[...], 0.0)
    h1 = jnp.maximum(s1_ref[...] * d + b2b_ref[...], 0.0)
    out = (jnp.dot(h0, w3a_ref[...],
                   preferred_element_type=jnp.float32, precision=lax.Precision.HIGHEST)
           + jnp.dot(h1, w3b_ref[...],
                     preferred_element_type=jnp.float32, precision=lax.Precision.HIGHEST)
           + b3_ref[...])
    out_ref[...] = out[:, 0]


_head_call = pl.pallas_call(
    _head_body,
    grid=(GRID,),
    in_specs=[
        pl.BlockSpec((R, 16), lambda i: (i, 0)),
        pl.BlockSpec((R, 16), lambda i: (i, 0)),
        pl.BlockSpec((2, R), lambda i: (0, i)),
        pl.BlockSpec((1, 16), lambda i: (0, 0)),
        pl.BlockSpec((1, 16), lambda i: (0, 0)),
        pl.BlockSpec((16, 1), lambda i: (0, 0)),
        pl.BlockSpec((16, 1), lambda i: (0, 0)),
        pl.BlockSpec((1, 1), lambda i: (0, 0)),
    ],
    out_specs=pl.BlockSpec((R,), lambda i: (i,)),
    out_shape=jax.ShapeDtypeStruct((NPAD,), jnp.float32),
)


def kernel(x, edge_index, W1, b1, W2, b2, W3, b3):
    src = edge_index[0].astype(jnp.int32)
    dst = edge_index[1].astype(jnp.int32)
    pad = jnp.full((NE_PAD - E,), NPAD - 1, jnp.int32)
    src2d = jnp.concatenate([src, pad]).reshape(NB, 128)
    dst2d = jnp.concatenate([dst, pad]).reshape(NB, 128)

    xpad = jnp.pad(x, ((0, NPAD - N), (0, 32 - x.shape[1])))
    zeros1d = jnp.zeros((NPAD,), jnp.float32)

    src1d = jnp.concatenate([src, pad])
    deg2 = _deg_kernel(dst2d, zeros1d)
    ya, yb = _prep_call(deg2, xpad)

    s0, s1 = _scatter_kernel(src1d, dst2d, ya, yb)

    w1p = jnp.pad(W1, ((0, 32 - W1.shape[0]), (0, 0)))
    y2a, y2b = _l1_call(s0, s1, deg2, w1p[:16], w1p[16:], b1[None, :], W2)

    t0, t1 = _scatter_kernel(src1d, dst2d, y2a, y2b)

    out = _head_call(t0, t1, deg2, b2[None, :16], b2[None, 16:],
                     W3[:16], W3[16:], b3[None, :])
    return out[:N, None]
